# baseline (speedup 1.0000x reference)
import functools

import jax
import jax.numpy as jnp
from jax import lax
from jax.experimental import pallas as pl
from jax.experimental.pallas import tpu as pltpu
from jax.experimental.pallas import tpu_sc as plsc

NC, NS = 2, 16
NW = NC * NS
BT = 128


def _router_body(hid_ref, gw_ref, ids_ref, ws_ref):
    x = hid_ref[...]
    gw = gw_ref[...]
    logits = lax.dot_general(
        x.astype(jnp.bfloat16), gw.astype(jnp.bfloat16), (((1,), (1,)), ((), ())),
        preferred_element_type=jnp.float32,
    )
    lane = lax.broadcasted_iota(jnp.int32, logits.shape, 1)
    m1 = jnp.max(logits, axis=1, keepdims=True)
    i1 = jnp.argmax(logits, axis=1)[:, None]
    masked = jnp.where(lane == i1, -jnp.inf, logits)
    m2 = jnp.max(masked, axis=1, keepdims=True)
    i2 = jnp.argmax(masked, axis=1)[:, None]
    e2 = jnp.exp(m2 - m1)
    w1 = 1.0 / (1.0 + e2)
    w2 = e2 / (1.0 + e2)
    ids_ref[...] = jnp.where(lane == 0, i1, jnp.where(lane == 1, i2, 0))
    ws_ref[...] = jnp.where(lane == 0, w1, jnp.where(lane == 1, w2, 0.0))


def _make_dispatch(n_exp, n_tok, n_pair, p_pad, nb_pad):
    mesh = plsc.VectorSubcoreMesh(core_axis_name="c", subcore_axis_name="s",
                                  num_cores=NC, num_subcores=NS)
    n_sc_chunks = n_pair // BT

    @functools.partial(
        pl.kernel, mesh=mesh,
        out_type=[
            jax.ShapeDtypeStruct((n_pair,), jnp.int32),
            jax.ShapeDtypeStruct((nb_pad,), jnp.int32),
            jax.ShapeDtypeStruct((p_pad,), jnp.float32),
        ],
        scratch_types=[
            pltpu.VMEM((n_pair,), jnp.int32),
            pltpu.VMEM((n_pair,), jnp.float32),
            pltpu.VMEM((n_pair,), jnp.int32),
            pltpu.VMEM((n_pair // BT, BT), jnp.int32),
            pltpu.VMEM((nb_pad,), jnp.int32),
            pltpu.VMEM((p_pad,), jnp.float32),
            pltpu.SemaphoreType.DMA,
        ],
    )
    def dispatch(eid_hbm, wcat_hbm, dest_hbm, be_hbm, wsort_hbm,
                 eid_v, wcat_v, dest_v, dest2_v, be_v, wz_v, sem):
        cid = lax.axis_index("c")
        sid = lax.axis_index("s")

        @pl.when((cid == 0) & (sid == 0))
        def _():
            pltpu.sync_copy(eid_hbm, eid_v)
            pltpu.sync_copy(wcat_hbm, wcat_v)

            def zinit(k, carry):
                wz_v[pl.ds(k * 16, 16)] = jnp.zeros((16,), jnp.float32)
                return carry

            lax.fori_loop(0, p_pad // 16, zinit, 0)
            pltpu.sync_copy(wz_v, wsort_hbm)

            def hist(i, accs):
                ev = eid_v[pl.ds(i * 16, 16)]
                one = jnp.ones((16,), jnp.int32)
                zero = jnp.zeros((16,), jnp.int32)
                return tuple(a + jnp.where(ev == e, one, zero)
                             for e, a in enumerate(accs))

            lanecnt = lax.fori_loop(
                0, n_pair // 16, hist,
                tuple(jnp.zeros((16,), jnp.int32) for _ in range(n_exp)))

            tots = []
            for e in range(n_exp):
                v = lanecnt[e]
                tot = v[0]
                for ln in range(1, 16):
                    tot = tot + v[ln]
                tots.append(tot)

            starts_s, ends_s = [], []
            s = tots[0] - tots[0]
            for e in range(n_exp):
                starts_s.append(s)
                s = s + jnp.bitwise_and(tots[e] + (BT - 1), -BT)
                ends_s.append(s)

            iota16 = lax.iota(jnp.int32, 16)
            zero16 = jnp.zeros((16,), jnp.int32)
            cur0 = []
            for e in range(n_exp):
                v = lanecnt[e]
                vec = zero16
                run = starts_s[e]
                for ln in range(16):
                    vec = jnp.where(iota16 == ln, zero16 + run, vec)
                    run = run + v[ln]
                cur0.append(vec)

            def rank(i, cur):
                ev = eid_v[pl.ds(i * 16, 16)]
                one = jnp.ones((16,), jnp.int32)
                zero = jnp.zeros((16,), jnp.int32)
                destv = zero
                newcur = []
                for e in range(n_exp):
                    m = ev == e
                    destv = jnp.where(m, cur[e], destv)
                    newcur.append(cur[e] + jnp.where(m, one, zero))
                dest_v[pl.ds(i * 16, 16)] = destv
                return tuple(newcur)

            lax.fori_loop(0, n_pair // 16, rank, tuple(cur0))

            for j in range(n_pair // BT):
                for k in range(BT // 16):
                    dest2_v[j, pl.ds(k * 16, 16)] = (
                        dest_v[pl.ds(j * BT + k * 16, 16)])

            for j in range(n_pair // BT):
                pltpu.async_copy(wcat_v.at[pl.ds(j * BT, BT)],
                                 wsort_hbm.at[dest2_v.at[j]], sem).wait()

            for vb in range(nb_pad // 16):
                bv = (lax.iota(jnp.int32, 16) + vb * 16) * BT
                acc = jnp.zeros((16,), jnp.int32)
                one = jnp.ones((16,), jnp.int32)
                zero = jnp.zeros((16,), jnp.int32)
                for e in range(n_exp):
                    acc = acc + jnp.where(bv >= (zero + ends_s[e]), one, zero)
                be_v[pl.ds(vb * 16, 16)] = acc

            pltpu.sync_copy(dest_v, dest_hbm)
            pltpu.sync_copy(be_v, be_hbm)

    return dispatch


def _make_disperse(n_tok, n_pair, p_pad, dw):
    mesh = plsc.VectorSubcoreMesh(core_axis_name="c", subcore_axis_name="s",
                                  num_cores=NC, num_subcores=NS)
    per_w = n_pair // NW
    half = per_w // 2

    @functools.partial(
        pl.kernel, mesh=mesh,
        out_type=jax.ShapeDtypeStruct((p_pad, dw), jnp.int32),
        scratch_types=[
            pltpu.VMEM((2, half), jnp.int32),
            pltpu.VMEM((half, dw), jnp.int32),
            pltpu.SemaphoreType.DMA,
        ],
    )
    def disperse(dest_hbm, hid_hbm, x_hbm, idx2, rows_v, sem):
        wid = lax.axis_index("s") * NC + lax.axis_index("c")
        pb = wid * per_w
        toko = pb - jnp.where(pb >= n_tok, jnp.int32(n_tok), jnp.int32(0))
        for c in range(2):
            pltpu.sync_copy(dest_hbm.at[pl.ds(pb + c * half, half)],
                            idx2.at[c])
            pltpu.sync_copy(hid_hbm.at[pl.ds(toko + c * half, half)], rows_v)
            pltpu.async_copy(rows_v, x_hbm.at[idx2.at[c]], sem).wait()

    return disperse


def _make_combine(n_tok, d_model):
    mesh = plsc.VectorSubcoreMesh(core_axis_name="c", subcore_axis_name="s",
                                  num_cores=NC, num_subcores=NS)
    per_w = n_tok // NW
    chunk = 16
    iters = per_w // chunk

    @functools.partial(
        pl.kernel, mesh=mesh,
        out_type=jax.ShapeDtypeStruct((n_tok, d_model), jnp.float32),
        scratch_types=[
            pltpu.VMEM((chunk,), jnp.int32),
            pltpu.VMEM((chunk, d_model), jnp.float32),
            pltpu.VMEM((chunk, d_model), jnp.float32),
            pltpu.SemaphoreType.DMA,
        ],
    )
    def combine(y_hbm, dest_hbm, out_hbm, idx_v, buf0, buf1, sem):
        wid = lax.axis_index("s") * NC + lax.axis_index("c")
        tb = wid * per_w
        for c in range(iters):
            base = tb + c * chunk
            pltpu.sync_copy(dest_hbm.at[pl.ds(base, chunk)], idx_v)
            pltpu.async_copy(y_hbm.at[idx_v], buf0, sem).wait()
            pltpu.sync_copy(dest_hbm.at[pl.ds(n_tok + base, chunk)], idx_v)
            pltpu.async_copy(y_hbm.at[idx_v], buf1, sem).wait()
            for r in range(chunk):
                def add_row(k, carry, r=r):
                    buf0[r, pl.ds(k * 16, 16)] = (
                        buf0[r, pl.ds(k * 16, 16)]
                        + buf1[r, pl.ds(k * 16, 16)])
                    return carry
                lax.fori_loop(0, d_model // 16, add_row, 0)
            pltpu.sync_copy(buf0, out_hbm.at[pl.ds(base, chunk)])

    return combine


def _ffn_body(d_ff, n_exp, be_smem, x_ref, w13_ref, w2_ref, ws_ref, y_ref):
    b = pl.program_id(0)

    @pl.when(be_smem[b] < n_exp)
    def _():
        x = x_ref[...]
        h = lax.dot_general(x, w13_ref[0], (((1,), (0,)), ((), ())),
                            preferred_element_type=jnp.float32)
        g = h[:, :d_ff]
        u = h[:, d_ff:]
        act = ((g * jax.nn.sigmoid(g)) * u).astype(jnp.bfloat16)
        p = lax.dot_general(act, w2_ref[0], (((1,), (0,)), ((), ())),
                            preferred_element_type=jnp.float32)
        y_ref[...] = p * ws_ref[0]


def kernel(hidden_states, gate_w, w13, w2):
    t, d_model = hidden_states.shape
    n_exp, _, d_ff2 = w13.shape
    d_ff = d_ff2 // 2
    n_pair = 2 * t
    nb = (n_pair + n_exp * (BT - 1)) // BT
    nb = nb + (nb % 2)
    nb_pad = ((nb + 15) // 16) * 16
    p_pad = nb * BT
    dw = d_model // 2

    ids, ws = pl.pallas_call(
        _router_body,
        out_shape=[jax.ShapeDtypeStruct((t, n_exp), jnp.int32),
                   jax.ShapeDtypeStruct((t, n_exp), jnp.float32)],
    )(hidden_states, gate_w)

    eid = jnp.concatenate([ids[:, 0], ids[:, 1]])
    wcat = jnp.concatenate([ws[:, 0], ws[:, 1]])

    dest, be, wsort = _make_dispatch(n_exp, t, n_pair, p_pad, nb_pad)(
        eid, wcat)

    hid_bf = hidden_states.astype(jnp.bfloat16)
    hid_i32 = lax.bitcast_convert_type(
        hid_bf.reshape(t, dw, 2), jnp.int32)

    x_i32 = _make_disperse(t, n_pair, p_pad, dw)(dest, hid_i32)
    x_bf = lax.bitcast_convert_type(x_i32, jnp.bfloat16).reshape(p_pad, d_model)

    w13_bf = w13.astype(jnp.bfloat16)
    w2_bf = w2.astype(jnp.bfloat16)
    ws3 = wsort.reshape(p_pad // BT, BT, 1)

    grid_spec = pltpu.PrefetchScalarGridSpec(
        num_scalar_prefetch=1,
        grid=(nb,),
        in_specs=[
            pl.BlockSpec((BT, d_model), lambda b, be: (b, 0)),
            pl.BlockSpec((1, d_model, d_ff2),
                         lambda b, be: (jnp.minimum(be[b], n_exp - 1), 0, 0)),
            pl.BlockSpec((1, d_ff, d_model),
                         lambda b, be: (jnp.minimum(be[b], n_exp - 1), 0, 0)),
            pl.BlockSpec((1, BT, 1), lambda b, be: (b, 0, 0)),
        ],
        out_specs=pl.BlockSpec((BT, d_model), lambda b, be: (b, 0)),
    )
    y = pl.pallas_call(
        functools.partial(_ffn_body, d_ff, n_exp),
        grid_spec=grid_spec,
        out_shape=jax.ShapeDtypeStruct((p_pad, d_model), jnp.float32),
    )(be, x_bf, w13_bf, w2_bf, ws3)

    out = _make_combine(t, d_model)(y, dest)
    return out


# speedup vs baseline: 1.7776x; 1.7776x over previous
import functools

import jax
import jax.numpy as jnp
from jax import lax
from jax.experimental import pallas as pl
from jax.experimental.pallas import tpu as pltpu
from jax.experimental.pallas import tpu_sc as plsc

NC, NS = 2, 16
NW = NC * NS
BT = 128


def _router_body(hid_ref, gw_ref, ids_ref, ws_ref):
    x = hid_ref[...]
    gw = gw_ref[...]
    logits = lax.dot_general(
        x.astype(jnp.bfloat16), gw.astype(jnp.bfloat16), (((1,), (1,)), ((), ())),
        preferred_element_type=jnp.float32,
    )
    lane = lax.broadcasted_iota(jnp.int32, logits.shape, 1)
    m1 = jnp.max(logits, axis=1, keepdims=True)
    i1 = jnp.argmax(logits, axis=1)[:, None]
    masked = jnp.where(lane == i1, -jnp.inf, logits)
    m2 = jnp.max(masked, axis=1, keepdims=True)
    i2 = jnp.argmax(masked, axis=1)[:, None]
    e2 = jnp.exp(m2 - m1)
    w1 = 1.0 / (1.0 + e2)
    w2 = e2 / (1.0 + e2)
    ids_ref[...] = jnp.where(lane == 0, i1, jnp.where(lane == 1, i2, 0))
    ws_ref[...] = jnp.where(lane == 0, w1, jnp.where(lane == 1, w2, 0.0))


def _make_dispatch(n_exp, n_tok, n_pair, p_pad, nb_pad):
    mesh = plsc.VectorSubcoreMesh(core_axis_name="c", subcore_axis_name="s",
                                  num_cores=NC, num_subcores=NS)
    n_sc_chunks = n_pair // BT

    @functools.partial(
        pl.kernel, mesh=mesh,
        out_type=[
            jax.ShapeDtypeStruct((n_pair,), jnp.int32),
            jax.ShapeDtypeStruct((nb_pad,), jnp.int32),
            jax.ShapeDtypeStruct((p_pad,), jnp.float32),
        ],
        scratch_types=[
            pltpu.VMEM((n_pair,), jnp.int32),
            pltpu.VMEM((n_pair,), jnp.float32),
            pltpu.VMEM((n_pair,), jnp.int32),
            pltpu.VMEM((n_pair // BT, BT), jnp.int32),
            pltpu.VMEM((nb_pad,), jnp.int32),
            pltpu.VMEM((p_pad,), jnp.float32),
            pltpu.SemaphoreType.DMA,
        ],
    )
    def dispatch(eid_hbm, wcat_hbm, dest_hbm, be_hbm, wsort_hbm,
                 eid_v, wcat_v, dest_v, dest2_v, be_v, wz_v, sem):
        cid = lax.axis_index("c")
        sid = lax.axis_index("s")

        @pl.when((cid == 0) & (sid == 0))
        def _():
            pltpu.sync_copy(eid_hbm, eid_v)
            pltpu.sync_copy(wcat_hbm, wcat_v)

            def zinit(k, carry):
                wz_v[pl.ds(k * 16, 16)] = jnp.zeros((16,), jnp.float32)
                return carry

            lax.fori_loop(0, p_pad // 16, zinit, 0)
            pltpu.sync_copy(wz_v, wsort_hbm)

            def hist(i, accs):
                ev = eid_v[pl.ds(i * 16, 16)]
                one = jnp.ones((16,), jnp.int32)
                zero = jnp.zeros((16,), jnp.int32)
                return tuple(a + jnp.where(ev == e, one, zero)
                             for e, a in enumerate(accs))

            lanecnt = lax.fori_loop(
                0, n_pair // 16, hist,
                tuple(jnp.zeros((16,), jnp.int32) for _ in range(n_exp)))

            tots = []
            for e in range(n_exp):
                v = lanecnt[e]
                tot = v[0]
                for ln in range(1, 16):
                    tot = tot + v[ln]
                tots.append(tot)

            starts_s, ends_s = [], []
            s = tots[0] - tots[0]
            for e in range(n_exp):
                starts_s.append(s)
                s = s + jnp.bitwise_and(tots[e] + (BT - 1), -BT)
                ends_s.append(s)

            iota16 = lax.iota(jnp.int32, 16)
            zero16 = jnp.zeros((16,), jnp.int32)
            cur0 = []
            for e in range(n_exp):
                v = lanecnt[e]
                vec = zero16
                run = starts_s[e]
                for ln in range(16):
                    vec = jnp.where(iota16 == ln, zero16 + run, vec)
                    run = run + v[ln]
                cur0.append(vec)

            def rank(i, cur):
                ev = eid_v[pl.ds(i * 16, 16)]
                one = jnp.ones((16,), jnp.int32)
                zero = jnp.zeros((16,), jnp.int32)
                destv = zero
                newcur = []
                for e in range(n_exp):
                    m = ev == e
                    destv = jnp.where(m, cur[e], destv)
                    newcur.append(cur[e] + jnp.where(m, one, zero))
                dest_v[pl.ds(i * 16, 16)] = destv
                return tuple(newcur)

            lax.fori_loop(0, n_pair // 16, rank, tuple(cur0))

            for j in range(n_pair // BT):
                for k in range(BT // 16):
                    dest2_v[j, pl.ds(k * 16, 16)] = (
                        dest_v[pl.ds(j * BT + k * 16, 16)])

            for j in range(n_pair // BT):
                pltpu.async_copy(wcat_v.at[pl.ds(j * BT, BT)],
                                 wsort_hbm.at[dest2_v.at[j]], sem).wait()

            for vb in range(nb_pad // 16):
                bv = (lax.iota(jnp.int32, 16) + vb * 16) * BT
                acc = jnp.zeros((16,), jnp.int32)
                one = jnp.ones((16,), jnp.int32)
                zero = jnp.zeros((16,), jnp.int32)
                for e in range(n_exp):
                    acc = acc + jnp.where(bv >= (zero + ends_s[e]), one, zero)
                be_v[pl.ds(vb * 16, 16)] = acc

            pltpu.sync_copy(dest_v, dest_hbm)
            pltpu.sync_copy(be_v, be_hbm)

    return dispatch


def _make_disperse(n_tok, n_pair, p_pad, d_model):
    mesh = plsc.VectorSubcoreMesh(core_axis_name="c", subcore_axis_name="s",
                                  num_cores=NC, num_subcores=NS)
    per_w = n_pair // NW
    chunk = 32
    iters = per_w // chunk

    @functools.partial(
        pl.kernel, mesh=mesh,
        out_type=jax.ShapeDtypeStruct((p_pad, d_model), jnp.float32),
        scratch_types=[
            pltpu.VMEM((iters, chunk), jnp.int32),
            pltpu.VMEM((chunk, d_model), jnp.float32),
            pltpu.SemaphoreType.DMA,
        ],
    )
    def disperse(dest_hbm, hid_hbm, x_hbm, idx2, rows_v, sem):
        wid = lax.axis_index("s") * NC + lax.axis_index("c")
        pb = wid * per_w
        toko = pb - jnp.where(pb >= n_tok, jnp.int32(n_tok), jnp.int32(0))
        for c in range(iters):
            pltpu.sync_copy(dest_hbm.at[pl.ds(pb + c * chunk, chunk)],
                            idx2.at[c])
            pltpu.sync_copy(hid_hbm.at[pl.ds(toko + c * chunk, chunk)],
                            rows_v)
            pltpu.async_copy(rows_v, x_hbm.at[idx2.at[c]], sem).wait()

    return disperse


def _make_combine(n_tok, d_model):
    mesh = plsc.VectorSubcoreMesh(core_axis_name="c", subcore_axis_name="s",
                                  num_cores=NC, num_subcores=NS)
    per_w = n_tok // NW
    chunk = 16
    iters = per_w // chunk

    @functools.partial(
        pl.kernel, mesh=mesh,
        out_type=jax.ShapeDtypeStruct((n_tok, d_model), jnp.float32),
        scratch_types=[
            pltpu.VMEM((chunk,), jnp.int32),
            pltpu.VMEM((chunk, d_model), jnp.float32),
            pltpu.VMEM((chunk, d_model), jnp.float32),
            pltpu.SemaphoreType.DMA,
        ],
    )
    def combine(y_hbm, dest_hbm, out_hbm, idx_v, buf0, buf1, sem):
        wid = lax.axis_index("s") * NC + lax.axis_index("c")
        tb = wid * per_w
        for c in range(iters):
            base = tb + c * chunk
            pltpu.sync_copy(dest_hbm.at[pl.ds(base, chunk)], idx_v)
            pltpu.async_copy(y_hbm.at[idx_v], buf0, sem).wait()
            pltpu.sync_copy(dest_hbm.at[pl.ds(n_tok + base, chunk)], idx_v)
            pltpu.async_copy(y_hbm.at[idx_v], buf1, sem).wait()
            for r in range(chunk):
                def add_row(k, carry, r=r):
                    for u in range(8):
                        o = k * 128 + u * 16
                        buf0[r, pl.ds(o, 16)] = (
                            buf0[r, pl.ds(o, 16)]
                            + buf1[r, pl.ds(o, 16)])
                    return carry
                lax.fori_loop(0, d_model // 128, add_row, 0)
            pltpu.sync_copy(buf0, out_hbm.at[pl.ds(base, chunk)])

    return combine


def _ffn_body(d_ff, n_exp, be_smem, x_ref, w13_ref, w2_ref, ws_ref, y_ref):
    b = pl.program_id(0)

    @pl.when(be_smem[b] < n_exp)
    def _():
        x = x_ref[...].astype(jnp.bfloat16)
        h = lax.dot_general(x, w13_ref[0], (((1,), (0,)), ((), ())),
                            preferred_element_type=jnp.float32)
        g = h[:, :d_ff]
        u = h[:, d_ff:]
        act = ((g * jax.nn.sigmoid(g)) * u).astype(jnp.bfloat16)
        p = lax.dot_general(act, w2_ref[0], (((1,), (0,)), ((), ())),
                            preferred_element_type=jnp.float32)
        y_ref[...] = p * ws_ref[0]


def kernel(hidden_states, gate_w, w13, w2):
    t, d_model = hidden_states.shape
    n_exp, _, d_ff2 = w13.shape
    d_ff = d_ff2 // 2
    n_pair = 2 * t
    nb = (n_pair + n_exp * (BT - 1)) // BT
    nb = nb + (nb % 2)
    nb_pad = ((nb + 15) // 16) * 16
    p_pad = nb * BT

    ids, ws = pl.pallas_call(
        _router_body,
        out_shape=[jax.ShapeDtypeStruct((t, n_exp), jnp.int32),
                   jax.ShapeDtypeStruct((t, n_exp), jnp.float32)],
    )(hidden_states, gate_w)

    eid = jnp.concatenate([ids[:, 0], ids[:, 1]])
    wcat = jnp.concatenate([ws[:, 0], ws[:, 1]])

    dest, be, wsort = _make_dispatch(n_exp, t, n_pair, p_pad, nb_pad)(
        eid, wcat)

    x_f32 = _make_disperse(t, n_pair, p_pad, d_model)(dest, hidden_states)

    w13_bf = w13.astype(jnp.bfloat16)
    w2_bf = w2.astype(jnp.bfloat16)
    ws3 = wsort.reshape(p_pad // BT, BT, 1)

    grid_spec = pltpu.PrefetchScalarGridSpec(
        num_scalar_prefetch=1,
        grid=(nb,),
        in_specs=[
            pl.BlockSpec((BT, d_model), lambda b, be: (b, 0)),
            pl.BlockSpec((1, d_model, d_ff2),
                         lambda b, be: (jnp.minimum(be[b], n_exp - 1), 0, 0)),
            pl.BlockSpec((1, d_ff, d_model),
                         lambda b, be: (jnp.minimum(be[b], n_exp - 1), 0, 0)),
            pl.BlockSpec((1, BT, 1), lambda b, be: (b, 0, 0)),
        ],
        out_specs=pl.BlockSpec((BT, d_model), lambda b, be: (b, 0)),
    )
    y = pl.pallas_call(
        functools.partial(_ffn_body, d_ff, n_exp),
        grid_spec=grid_spec,
        out_shape=jax.ShapeDtypeStruct((p_pad, d_model), jnp.float32),
    )(be, x_f32, w13_bf, w2_bf, ws3)

    out = _make_combine(t, d_model)(y, dest)
    return out
